# Initial kernel scaffold; baseline (speedup 1.0000x reference)
import jax
import jax.numpy as jnp
from jax.experimental import pallas as pl


def kernel(input_text, embed_table):
    raise NotImplementedError("write your pallas kernel here")


# speedup vs baseline: 1.4596x; 1.4596x over previous
import functools

import jax
import jax.numpy as jnp
from jax import lax
from jax.experimental import pallas as pl
from jax.experimental.pallas import tpu as pltpu
from jax.experimental.pallas import tpu_sc as plsc

BLK = 128
K = 8


@functools.cache
def _build(n_blocks: int, d: int, vocab: int):
    info = plsc.get_sparse_core_info()
    nw = info.num_cores * info.num_subcores
    bpw = n_blocks // nw
    k = K if bpw % K == 0 else 1
    steps = bpw // k

    mesh = plsc.VectorSubcoreMesh(core_axis_name="c", subcore_axis_name="s")

    @functools.partial(
        pl.kernel,
        mesh=mesh,
        out_type=jax.ShapeDtypeStruct((n_blocks, BLK, d), jnp.float32),
        scratch_types=[
            pltpu.VMEM((k, BLK), jnp.int32),
            pltpu.VMEM((k, BLK, d), jnp.float32),
            pltpu.SemaphoreType.DMA,
        ],
        compiler_params=pltpu.CompilerParams(use_tc_tiling_on_sc=False),
    )
    def gather_kernel(idx_hbm, table_hbm, out_hbm, idx_v, rows_v, sem):
        wid = lax.axis_index("s") * info.num_cores + lax.axis_index("c")
        wstart = wid * bpw

        def step(g, carry):
            blk0 = wstart + g * k
            pltpu.sync_copy(idx_hbm.at[pl.ds(blk0, k)], idx_v)
            handles = [
                pltpu.async_copy(table_hbm.at[idx_v.at[j]], rows_v.at[j], sem)
                for j in range(k)
            ]
            for h in handles:
                h.wait()
            pltpu.sync_copy(rows_v, out_hbm.at[pl.ds(blk0, k)])
            return carry

        lax.fori_loop(0, steps, step, 0)

    return gather_kernel


def kernel(input_text, embed_table):
    batch, seq = input_text.shape
    vocab, d = embed_table.shape
    n = batch * seq
    idx = input_text.reshape(-1).astype(jnp.int32)

    nw = 32
    pad = (-n) % (BLK * nw)
    if pad:
        idx = jnp.concatenate([idx, jnp.zeros((pad,), jnp.int32)])
    n_blocks = (n + pad) // BLK

    out = _build(n_blocks, d, vocab)(idx.reshape(n_blocks, BLK), embed_table)
    return out.reshape(-1, d)[:n].reshape(batch, seq, d)


# speedup vs baseline: 1.4889x; 1.0201x over previous
import functools

import jax
import jax.numpy as jnp
from jax import lax
from jax.experimental import pallas as pl
from jax.experimental.pallas import tpu as pltpu
from jax.experimental.pallas import tpu_sc as plsc

BLK = 128


@functools.cache
def _build(n_blocks: int, d: int):
    info = plsc.get_sparse_core_info()
    nw = info.num_cores * info.num_subcores
    bpw = n_blocks // nw

    k = next(c for c in (10, 8, 5, 4, 2, 1)
             if bpw % c == 0 and (bpw // c) % 2 == 0)
    steps = bpw // k

    mesh = plsc.VectorSubcoreMesh(core_axis_name="c", subcore_axis_name="s")

    @functools.partial(
        pl.kernel,
        mesh=mesh,
        out_type=jax.ShapeDtypeStruct((n_blocks, BLK, d), jnp.float32),
        scratch_types=[
            pltpu.VMEM((2, k, BLK), jnp.int32),
            pltpu.VMEM((2, k, BLK, d), jnp.float32),
            pltpu.SemaphoreType.DMA,
            pltpu.SemaphoreType.DMA,
        ],
        compiler_params=pltpu.CompilerParams(use_tc_tiling_on_sc=False),
    )
    def gather_kernel(idx_hbm, table_hbm, out_hbm, idx_v, rows_v, sem0, sem1):
        wid = lax.axis_index("s") * info.num_cores + lax.axis_index("c")
        wstart = wid * bpw
        sems = (sem0, sem1)

        def load_and_fire(g, b):
            blk0 = wstart + g * k
            pltpu.sync_copy(idx_hbm.at[pl.ds(blk0, k)], idx_v.at[b])
            for j in range(k):
                pltpu.async_copy(
                    table_hbm.at[idx_v.at[b].at[j]], rows_v.at[b].at[j], sems[b]
                )

        def drain_and_store(g, b):
            blk0 = wstart + g * k
            pltpu.make_async_copy(
                out_hbm.at[pl.ds(blk0, k)], rows_v.at[b], sems[b]
            ).wait()
            pltpu.sync_copy(rows_v.at[b], out_hbm.at[pl.ds(blk0, k)])

        load_and_fire(0, 0)
        load_and_fire(1, 1)

        def body(t, carry):
            for b in (0, 1):
                g = 2 * t + b
                drain_and_store(g, b)
                load_and_fire(g + 2, b)
            return carry

        lax.fori_loop(0, steps // 2 - 1, body, 0)

        drain_and_store(steps - 2, 0)
        drain_and_store(steps - 1, 1)

    return gather_kernel


def kernel(input_text, embed_table):
    batch, seq = input_text.shape
    vocab, d = embed_table.shape
    n = batch * seq
    idx = input_text.reshape(-1).astype(jnp.int32)

    nw = 32
    pad = (-n) % (BLK * nw * 2)
    if pad:
        idx = jnp.concatenate([idx, jnp.zeros((pad,), jnp.int32)])
    n_blocks = (n + pad) // BLK

    out = _build(n_blocks, d)(idx.reshape(n_blocks, BLK), embed_table)
    return out.reshape(-1, d)[:n].reshape(batch, seq, d)


# speedup vs baseline: 2.0731x; 1.3924x over previous
import functools

import jax
import jax.numpy as jnp
from jax import lax
from jax.experimental import pallas as pl
from jax.experimental.pallas import tpu as pltpu
from jax.experimental.pallas import tpu_sc as plsc

BLK = 128
TC = 4096


@functools.cache
def _build_gather(n_blocks: int, d: int):
    info = plsc.get_sparse_core_info()
    nw = info.num_cores * info.num_subcores
    bpw = n_blocks // nw

    k = next(c for c in (10, 8, 5, 4, 2, 1)
             if bpw % c == 0 and (bpw // c) % 2 == 0)
    steps = bpw // k

    mesh = plsc.VectorSubcoreMesh(core_axis_name="c", subcore_axis_name="s")

    @functools.partial(
        pl.kernel,
        mesh=mesh,
        out_type=jax.ShapeDtypeStruct((n_blocks, BLK, d), jnp.float32),
        scratch_types=[
            pltpu.VMEM((2, k, BLK), jnp.int32),
            pltpu.VMEM((2, k, BLK, d), jnp.float32),
            pltpu.SemaphoreType.DMA,
            pltpu.SemaphoreType.DMA,
        ],
        compiler_params=pltpu.CompilerParams(use_tc_tiling_on_sc=False),
    )
    def gather_kernel(idx_hbm, table_hbm, out_hbm, idx_v, rows_v, sem0, sem1):
        wid = lax.axis_index("s") * info.num_cores + lax.axis_index("c")
        wstart = wid * bpw
        sems = (sem0, sem1)

        def load_and_fire(g, b):
            blk0 = wstart + g * k
            pltpu.sync_copy(idx_hbm.at[pl.ds(blk0, k)], idx_v.at[b])
            for j in range(k):
                pltpu.async_copy(
                    table_hbm.at[idx_v.at[b].at[j]], rows_v.at[b].at[j], sems[b]
                )

        def drain_and_store(g, b):
            blk0 = wstart + g * k
            pltpu.make_async_copy(
                out_hbm.at[pl.ds(blk0, k)], rows_v.at[b], sems[b]
            ).wait()
            pltpu.sync_copy(rows_v.at[b], out_hbm.at[pl.ds(blk0, k)])

        load_and_fire(0, 0)
        load_and_fire(1, 1)

        def body(t, carry):
            for b in (0, 1):
                g = 2 * t + b
                drain_and_store(g, b)
                load_and_fire(g + 2, b)
            return carry

        lax.fori_loop(0, steps // 2 - 1, body, 0)

        drain_and_store(steps - 2, 0)
        drain_and_store(steps - 1, 1)

    return gather_kernel


@functools.cache
def _build_table_t(v: int, d: int):
    g = 128 // d
    span = g * TC
    nb = -(-v // span)

    def body(in_ref, out_ref):
        x = in_ref[...]
        out_ref[...] = jnp.concatenate(
            [x[:, q * TC:(q + 1) * TC].T for q in range(g)], axis=1
        )

    call = pl.pallas_call(
        body,
        grid=(nb,),
        in_specs=[pl.BlockSpec((d, span), lambda i: (0, i))],
        out_specs=pl.BlockSpec((TC, 128), lambda i: (i, 0)),
        out_shape=jax.ShapeDtypeStruct((nb * TC, 128), jnp.float32),
    )
    return call, nb * span


@functools.cache
def _build_out_t(s: int, b: int, d: int):
    g = 128 // d
    rows = b // g

    def body(in_ref, out_ref):
        x = in_ref[...]
        out_ref[0] = jnp.concatenate(
            [x[:, q * d:(q + 1) * d].T for q in range(g)], axis=1
        )

    return pl.pallas_call(
        body,
        grid=(s,),
        in_specs=[pl.BlockSpec((rows, 128), lambda i: (i, 0))],
        out_specs=pl.BlockSpec((1, d, b), lambda i: (i, 0, 0)),
        out_shape=jax.ShapeDtypeStruct((s, d, b), jnp.float32),
    )


def kernel(input_text, embed_table):
    batch, seq = input_text.shape
    vocab, d = embed_table.shape
    n = batch * seq
    nw = 32
    g = 128 // d if 128 % d == 0 else 0

    fast = (
        g > 0
        and n % (BLK * nw * 2) == 0
        and batch % g == 0
        and (batch // g) % 8 == 0
    )

    if fast:
        idx = (
            input_text.T.reshape(seq, g, batch // g)
            .transpose(0, 2, 1)
            .reshape(-1)
            .astype(jnp.int32)
        )
    else:
        idx = input_text.reshape(-1).astype(jnp.int32)

    if g:
        span = g * TC
        blk_i = idx // span
        rem = idx % span
        idx = (blk_i * TC + rem % TC) * g + rem // TC
        t_call, v_pad = _build_table_t(vocab, d)
        table_pk = t_call(embed_table.T).reshape(v_pad, d)
    else:
        table_pk = embed_table

    pad = (-n) % (BLK * nw * 2)
    if pad:
        idx = jnp.concatenate([idx, jnp.zeros((pad,), jnp.int32)])
    n_blocks = (n + pad) // BLK
    gth = _build_gather(n_blocks, d)(idx.reshape(n_blocks, BLK), table_pk)

    if fast:
        p = _build_out_t(seq, batch, d)(gth.reshape(n_blocks * BLK // g, 128))
        return jnp.transpose(p, (2, 0, 1))
    return gth.reshape(-1, d)[:n].reshape(batch, seq, d)


# speedup vs baseline: 3.5462x; 1.7105x over previous
import functools

import jax
import jax.numpy as jnp
from jax import lax
from jax.experimental import pallas as pl
from jax.experimental.pallas import tpu as pltpu
from jax.experimental.pallas import tpu_sc as plsc

BLK = 128
TC = 4096


@functools.cache
def _build_gather(n_blocks: int, d: int):
    info = plsc.get_sparse_core_info()
    nw = info.num_cores * info.num_subcores
    bpw = n_blocks // nw

    k = next(c for c in (10, 8, 5, 4, 2, 1)
             if bpw % c == 0 and (bpw // c) % 2 == 0)
    steps = bpw // k

    mesh = plsc.VectorSubcoreMesh(core_axis_name="c", subcore_axis_name="s")

    @functools.partial(
        pl.kernel,
        mesh=mesh,
        out_type=jax.ShapeDtypeStruct((n_blocks, BLK, d), jnp.float32),
        scratch_types=[
            pltpu.VMEM((2, k, BLK), jnp.int32),
            pltpu.VMEM((2, k, BLK, d), jnp.float32),
            pltpu.SemaphoreType.DMA,
            pltpu.SemaphoreType.DMA,
        ],
        compiler_params=pltpu.CompilerParams(use_tc_tiling_on_sc=False),
    )
    def gather_kernel(idx_hbm, table_hbm, out_hbm, idx_v, rows_v, sem0, sem1):
        wid = lax.axis_index("s") * info.num_cores + lax.axis_index("c")
        wstart = wid * bpw
        sems = (sem0, sem1)

        def load_and_fire(g, b):
            blk0 = wstart + g * k
            pltpu.sync_copy(idx_hbm.at[pl.ds(blk0, k)], idx_v.at[b])
            for j in range(k):
                pltpu.async_copy(
                    table_hbm.at[idx_v.at[b].at[j]], rows_v.at[b].at[j], sems[b]
                )

        def drain_and_store(g, b):
            blk0 = wstart + g * k
            pltpu.make_async_copy(
                out_hbm.at[pl.ds(blk0, k)], rows_v.at[b], sems[b]
            ).wait()
            pltpu.sync_copy(rows_v.at[b], out_hbm.at[pl.ds(blk0, k)])

        load_and_fire(0, 0)
        load_and_fire(1, 1)

        def body(t, carry):
            for b in (0, 1):
                g = 2 * t + b
                drain_and_store(g, b)
                load_and_fire(g + 2, b)
            return carry

        lax.fori_loop(0, steps // 2 - 1, body, 0)

        drain_and_store(steps - 2, 0)
        drain_and_store(steps - 1, 1)

    return gather_kernel


@functools.cache
def _build_table_t(v: int, d: int):
    g = 128 // d
    span = g * TC
    nb = -(-v // span)

    dn = (((0,), (0,)), ((), ()))

    def body(in_ref, out_ref):
        x = in_ref[...]
        xs = jnp.concatenate(
            [x[:, q * TC:(q + 1) * TC] for q in range(g)], axis=0
        )
        out_ref[...] = lax.dot_general(
            xs, jnp.eye(128, 128, dtype=jnp.float32), dn,
            preferred_element_type=jnp.float32,
        )

    call = pl.pallas_call(
        body,
        grid=(nb,),
        in_specs=[pl.BlockSpec((d, span), lambda i: (0, i))],
        out_specs=pl.BlockSpec((TC, 128), lambda i: (i, 0)),
        out_shape=jax.ShapeDtypeStruct((nb * TC, 128), jnp.float32),
        compiler_params=pltpu.CompilerParams(
            fuse_transposed_lhs_in_matmul=True
        ),
    )
    return call, nb * span


@functools.cache
def _build_out_t(s: int, b: int, d: int):
    g = 128 // d
    rows = b // g
    sb = next(c for c in (8, 4, 2, 1) if s % c == 0)

    dn = (((0,), (1,)), ((), ()))

    def body(in_ref, out_ref):
        eye = jnp.eye(d, d, dtype=jnp.float32)
        for ss in range(sb):
            x = in_ref[pl.ds(ss * rows, rows)]
            out_ref[ss] = jnp.concatenate(
                [
                    lax.dot_general(eye, x[:, q * d:(q + 1) * d], dn,
                                    preferred_element_type=jnp.float32)
                    for q in range(g)
                ],
                axis=1,
            )

    return pl.pallas_call(
        body,
        grid=(s // sb,),
        in_specs=[pl.BlockSpec((sb * rows, 128), lambda i: (i, 0))],
        out_specs=pl.BlockSpec((sb, d, b), lambda i: (i, 0, 0)),
        out_shape=jax.ShapeDtypeStruct((s, d, b), jnp.float32),
    )


def kernel(input_text, embed_table):
    batch, seq = input_text.shape
    vocab, d = embed_table.shape
    n = batch * seq
    nw = 32
    g = 128 // d if 128 % d == 0 else 0

    fast = (
        g > 0
        and n % (BLK * nw * 2) == 0
        and batch % g == 0
        and (batch // g) % 8 == 0
    )

    if fast:
        idx = (
            input_text.T.reshape(seq, g, batch // g)
            .transpose(0, 2, 1)
            .reshape(-1)
            .astype(jnp.int32)
        )
    else:
        idx = input_text.reshape(-1).astype(jnp.int32)

    if g:
        span = g * TC
        blk_i = idx // span
        rem = idx % span
        idx = (blk_i * TC + rem % TC) * g + rem // TC
        t_call, v_pad = _build_table_t(vocab, d)
        table_pk = t_call(embed_table.T).reshape(v_pad, d)
    else:
        table_pk = embed_table

    pad = (-n) % (BLK * nw * 2)
    if pad:
        idx = jnp.concatenate([idx, jnp.zeros((pad,), jnp.int32)])
    n_blocks = (n + pad) // BLK
    gth = _build_gather(n_blocks, d)(idx.reshape(n_blocks, BLK), table_pk)

    if fast:
        p = _build_out_t(seq, batch, d)(gth.reshape(n_blocks * BLK // g, 128))
        return jnp.transpose(p, (2, 0, 1))
    return gth.reshape(-1, d)[:n].reshape(batch, seq, d)


# speedup vs baseline: 5.2376x; 1.4770x over previous
import functools

import jax
import jax.numpy as jnp
from jax import lax
from jax.experimental import pallas as pl
from jax.experimental.pallas import tpu as pltpu
from jax.experimental.pallas import tpu_sc as plsc

BLK = 128
TC = 4096


@functools.cache
def _build_gather(n_blocks: int, d: int, batch: int, permute: bool):
    info = plsc.get_sparse_core_info()
    nw = info.num_cores * info.num_subcores
    bpw = n_blocks // nw

    if permute:
        g = 128 // d
        r = batch // BLK
        rq = r // g
        k = g
        h = d // 16
    else:
        k = next(c for c in (10, 8, 5, 4, 2, 1)
                 if bpw % c == 0 and (bpw // c) % 2 == 0)
    steps = bpw // k

    mesh = plsc.VectorSubcoreMesh(core_axis_name="c", subcore_axis_name="s")

    @functools.partial(
        pl.kernel,
        mesh=mesh,
        out_type=jax.ShapeDtypeStruct((n_blocks, BLK, d), jnp.float32),
        scratch_types=[
            pltpu.VMEM((2, k, BLK), jnp.int32),
            pltpu.VMEM((2, k, BLK, d), jnp.float32),
            pltpu.VMEM((2, k, BLK), jnp.int32),
            pltpu.SemaphoreType.DMA,
            pltpu.SemaphoreType.DMA,
            pltpu.SemaphoreType.DMA,
            pltpu.SemaphoreType.DMA,
        ],
        compiler_params=pltpu.CompilerParams(
            use_tc_tiling_on_sc=False, needs_layout_passes=False
        ),
    )
    def gather_kernel(idx_hbm, table_hbm, out_hbm, idx_v, rows_v, seg_v,
                      sem0, sem1, ssem0, ssem1):
        wid = lax.axis_index("s") * info.num_cores + lax.axis_index("c")
        wstart = wid * bpw
        sems = (sem0, sem1)
        ssems = (ssem0, ssem1)

        if permute:
            iota16 = lax.iota(jnp.int32, 16)

            def fire_seg(gs, b):
                e = jnp.minimum(wstart // k + gs, n_blocks // k - 1)
                s = e // rq
                kap = e % rq
                row0 = s * r + kap
                for q in range(g):
                    pltpu.async_copy(
                        idx_hbm.at[pl.ds(row0 + q * rq, 1)],
                        seg_v.at[b].at[pl.ds(q, 1)],
                        ssems[b],
                    )

            def wait_seg(gs, b):
                e = jnp.minimum(wstart // k + gs, n_blocks // k - 1)
                s = e // rq
                kap = e % rq
                row0 = s * r + kap
                for q in range(g):
                    pltpu.make_async_copy(
                        idx_hbm.at[pl.ds(row0 + q * rq, 1)],
                        seg_v.at[b].at[pl.ds(q, 1)],
                        ssems[b],
                    ).wait()

            def build_idx(b):
                for jj in range(g):
                    dst = idx_v.at[b].at[jj]
                    for q in range(g):
                        for hh in range(h):
                            vals = seg_v[b, q, pl.ds(jj * d + hh * 16, 16)]
                            pos = (iota16 + hh * 16) * g + q
                            plsc.store_scatter(dst, [pos], vals)
        else:
            def fire_seg(gs, b):
                pass

            def wait_seg(gs, b):
                pass

            def build_idx(b):
                pass

        def load_and_fire(gs, b):
            if permute:
                wait_seg(gs, b)
                build_idx(b)
            else:
                pltpu.sync_copy(
                    idx_hbm.at[pl.ds(wstart + gs * k, k)], idx_v.at[b]
                )
            for j in range(k):
                pltpu.async_copy(
                    table_hbm.at[idx_v.at[b].at[j]], rows_v.at[b].at[j],
                    sems[b],
                )
            if permute:
                fire_seg(gs + 2, b)

        def drain_and_store(gs, b):
            blk0 = wstart + gs * k
            pltpu.make_async_copy(
                out_hbm.at[pl.ds(blk0, k)], rows_v.at[b], sems[b]
            ).wait()
            pltpu.sync_copy(rows_v.at[b], out_hbm.at[pl.ds(blk0, k)])

        if permute:
            fire_seg(0, 0)
            fire_seg(1, 1)
        load_and_fire(0, 0)
        load_and_fire(1, 1)

        def body(t, carry):
            for b in (0, 1):
                gs = 2 * t + b
                drain_and_store(gs, b)
                load_and_fire(gs + 2, b)
            return carry

        lax.fori_loop(0, steps // 2 - 1, body, 0)

        drain_and_store(steps - 2, 0)
        drain_and_store(steps - 1, 1)
        if permute:
            wait_seg(steps, steps % 2)
            wait_seg(steps + 1, (steps + 1) % 2)

    return gather_kernel


@functools.cache
def _build_table_t(v: int, d: int):
    g = 128 // d
    span = g * TC
    nb = -(-v // span)

    dn = (((0,), (0,)), ((), ()))

    def body(in_ref, out_ref):
        x = in_ref[...]
        xs = jnp.concatenate(
            [x[:, q * TC:(q + 1) * TC] for q in range(g)], axis=0
        )
        out_ref[...] = lax.dot_general(
            xs, jnp.eye(128, 128, dtype=jnp.float32), dn,
            preferred_element_type=jnp.float32,
        )

    call = pl.pallas_call(
        body,
        grid=(nb,),
        in_specs=[pl.BlockSpec((d, span), lambda i: (0, i))],
        out_specs=pl.BlockSpec((TC, 128), lambda i: (i, 0)),
        out_shape=jax.ShapeDtypeStruct((nb * TC, 128), jnp.float32),
        compiler_params=pltpu.CompilerParams(
            fuse_transposed_lhs_in_matmul=True
        ),
    )
    return call, nb * span


@functools.cache
def _build_out_t(s: int, b: int, d: int):
    g = 128 // d
    rows = b // g
    sb = next(c for c in (8, 4, 2, 1) if s % c == 0)

    dn = (((0,), (1,)), ((), ()))

    def body(in_ref, out_ref):
        eye = jnp.eye(d, d, dtype=jnp.float32)
        for ss in range(sb):
            x = in_ref[pl.ds(ss * rows, rows)]
            out_ref[ss] = jnp.concatenate(
                [
                    lax.dot_general(eye, x[:, q * d:(q + 1) * d], dn,
                                    preferred_element_type=jnp.float32)
                    for q in range(g)
                ],
                axis=1,
            )

    return pl.pallas_call(
        body,
        grid=(s // sb,),
        in_specs=[pl.BlockSpec((sb * rows, 128), lambda i: (i, 0))],
        out_specs=pl.BlockSpec((sb, d, b), lambda i: (i, 0, 0)),
        out_shape=jax.ShapeDtypeStruct((s, d, b), jnp.float32),
    )


def kernel(input_text, embed_table):
    batch, seq = input_text.shape
    vocab, d = embed_table.shape
    n = batch * seq
    nw = 32
    g = 128 // d if 128 % d == 0 else 0

    fast = (
        g > 0
        and d % 16 == 0
        and batch % BLK == 0
        and (batch // BLK) % g == 0
        and n % (BLK * g * nw * 2) == 0
    )

    if fast:
        idx = input_text.T.reshape(-1).astype(jnp.int32)
    else:
        idx = input_text.reshape(-1).astype(jnp.int32)

    if g:
        span = g * TC
        blk_i = idx // span
        rem = idx % span
        idx = (blk_i * TC + rem % TC) * g + rem // TC
        t_call, v_pad = _build_table_t(vocab, d)
        table_pk = t_call(embed_table.T).reshape(v_pad, d)
    else:
        table_pk = embed_table

    pad = (-n) % (BLK * nw * 2)
    if pad:
        idx = jnp.concatenate([idx, jnp.zeros((pad,), jnp.int32)])
    n_blocks = (n + pad) // BLK

    gth = _build_gather(n_blocks, d, batch, fast)(
        idx.reshape(n_blocks, BLK), table_pk
    )

    if fast:
        p = _build_out_t(seq, batch, d)(gth.reshape(n_blocks * BLK // g, 128))
        return jnp.transpose(p, (2, 0, 1))
    return gth.reshape(-1, d)[:n].reshape(batch, seq, d)
